# Initial kernel scaffold; baseline (speedup 1.0000x reference)
import jax
import jax.numpy as jnp
from jax.experimental import pallas as pl


def kernel(x, edge_attr, edge_index, enc_We, enc_be, enc_Wn, enc_bn, proc_We, proc_be, proc_Wn, proc_bn, dec_We, dec_be, dec_Wn, dec_bn):
    raise NotImplementedError("write your pallas kernel here")


# speedup vs baseline: 4.0386x; 4.0386x over previous
import functools

import jax
import jax.numpy as jnp
from jax import lax
from jax.experimental import pallas as pl
from jax.experimental.pallas import tpu as pltpu
from jax.experimental.pallas import tpu_sc as plsc

N = 10000
E = 320000
DF = 128
DE = 16
L = 32

NC = 2
NS = 16
NW = NC * NS
EW = E // NW
CH = 80
NCH = EW // CH
NP = 10240
NROWS = NP // NS

E4 = E // 4


def _make_sc_block(write_e: bool):
    mesh = plsc.VectorSubcoreMesh(
        core_axis_name="c", subcore_axis_name="s", num_cores=NC, num_subcores=NS
    )
    out_type = []
    if write_e:
        out_type.append(jax.ShapeDtypeStruct((NW, EW, L), jnp.float32))
    out_type.append(jax.ShapeDtypeStruct((NC, NP, L), jnp.float32))

    scratch = [
        pltpu.VMEM((NCH, CH), jnp.int32),
        pltpu.VMEM((NCH, CH), jnp.int32),
        pltpu.VMEM((CH, L), jnp.float32),
        pltpu.VMEM((CH, L), jnp.float32),
        pltpu.VMEM((CH, L), jnp.float32),
        pltpu.VMEM((CH, L), jnp.float32),
        pltpu.SemaphoreType.DMA,
        pltpu.VMEM_SHARED((NP, L), jnp.float32),
    ]

    def body(s_hbm, r_hbm, base_hbm, S_hbm, R_hbm, z_hbm, *rest):
        if write_e:
            e_out, agg_out = rest[0], rest[1]
            scr = rest[2:]
        else:
            agg_out = rest[0]
            scr = rest[1:]
        s_v, r_v, b_v, sr_v, rr_v, e_v, sem, agg_sh = scr

        c = lax.axis_index("c")
        sid = lax.axis_index("s")
        t = c * NS + sid

        pltpu.sync_copy(
            z_hbm.at[pl.ds(sid * NROWS, NROWS)],
            agg_sh.at[pl.ds(sid * NROWS, NROWS)],
        )
        pltpu.sync_copy(s_hbm.at[t], s_v)
        pltpu.sync_copy(r_hbm.at[t], r_v)
        plsc.subcore_barrier()

        @pl.loop(0, NCH)
        def _chunk(j):
            pltpu.sync_copy(base_hbm.at[t, pl.ds(j * CH, CH)], b_v)
            d1 = pltpu.async_copy(S_hbm.at[s_v.at[j]], sr_v, sem)
            d2 = pltpu.async_copy(R_hbm.at[r_v.at[j]], rr_v, sem)
            d1.wait()
            d2.wait()

            @pl.loop(0, CH, unroll=8)
            def _row(i):
                for h in range(2):
                    sl = pl.ds(h * 16, 16)
                    e_v[i, sl] = jnp.maximum(
                        b_v[i, sl] + sr_v[i, sl] + rr_v[i, sl], 0.0
                    )

            if write_e:
                pltpu.sync_copy(e_v, e_out.at[t, pl.ds(j * CH, CH)])
            pltpu.sync_copy(e_v, agg_sh.at[r_v.at[j]], add=True)

        plsc.subcore_barrier()
        pltpu.sync_copy(
            agg_sh.at[pl.ds(sid * NROWS, NROWS)],
            agg_out.at[c, pl.ds(sid * NROWS, NROWS)],
        )

    return pl.kernel(
        body,
        out_type=tuple(out_type),
        mesh=mesh,
        scratch_types=scratch,
        compiler_params=pltpu.CompilerParams(use_tc_tiling_on_sc=False),
    )


_sc_block_we = _make_sc_block(True)
_sc_block_agg = _make_sc_block(False)


def _dense(ins, outs_spec, block_rows):
    rows = ins[0].shape[0]
    grid = rows // block_rows
    n_in = len(ins)
    w_arrays = []
    w_pos = []
    for terms, _, _ in outs_spec:
        pos = []
        for _, _, W in terms:
            pos.append(len(w_arrays))
            w_arrays.append(W)
        w_pos.append(pos)
    biases = [b.reshape(1, -1) for _, b, _ in outs_spec]
    n_w = len(w_arrays)
    n_out = len(outs_spec)

    def body(*refs):
        in_refs = refs[:n_in]
        w_refs = refs[n_in:n_in + n_w]
        b_refs = refs[n_in + n_w:n_in + n_w + n_out]
        o_refs = refs[n_in + n_w + n_out:]
        outvals = []
        for j, (terms, _, relu) in enumerate(outs_spec):
            acc = b_refs[j][...]
            for (kind, idx, _), wp in zip(terms, w_pos[j]):
                op = in_refs[idx][...] if kind == "in" else outvals[idx]
                acc = acc + jnp.dot(
                    op, w_refs[wp][...],
                    preferred_element_type=jnp.float32,
                    precision=jax.lax.Precision.HIGHEST,
                )
            val = jnp.maximum(acc, 0.0) if relu else acc
            outvals.append(val)
            o_refs[j][...] = val

    in_specs = (
        [pl.BlockSpec((block_rows, a.shape[1]), lambda i: (i, 0)) for a in ins]
        + [pl.BlockSpec(w.shape, lambda i: (0, 0)) for w in w_arrays]
        + [pl.BlockSpec(b.shape, lambda i: (0, 0)) for b in biases]
    )
    out_specs = [
        pl.BlockSpec((block_rows, b.shape[1]), lambda i: (i, 0)) for b in biases
    ]
    out_shape = [
        jax.ShapeDtypeStruct((rows, b.shape[1]), jnp.float32) for b in biases
    ]
    res = pl.pallas_call(
        body,
        grid=(grid,),
        in_specs=in_specs,
        out_specs=out_specs,
        out_shape=out_shape,
    )(*ins, *w_arrays, *biases)
    return res


def _blockdiag(W, k):
    din, dout = W.shape
    out = jnp.zeros((k * din, k * dout), jnp.float32)
    for i in range(k):
        out = out.at[i * din:(i + 1) * din, i * dout:(i + 1) * dout].set(W)
    return out


def kernel(x, edge_attr, edge_index,
           enc_We, enc_be, enc_Wn, enc_bn,
           proc_We, proc_be, proc_Wn, proc_bn,
           dec_We, dec_be, dec_Wn, dec_bn):
    s3 = edge_index[0].reshape(NW, NCH, CH)
    r3 = edge_index[1].reshape(NW, NCH, CH)
    zeros = jnp.zeros((NP, L), jnp.float32)

    We_e = enc_We[:DE]
    We_s = enc_We[DE:DE + DF]
    We_r = enc_We[DE + DF:]
    W_ce = proc_We[0 * L:1 * L]
    W_ee = proc_We[1 * L:2 * L]
    W_scx = proc_We[2 * L:3 * L]
    W_sex = proc_We[3 * L:4 * L]
    W_rcx = proc_We[4 * L:5 * L]
    W_rex = proc_We[5 * L:6 * L]
    Wn_cx = proc_Wn[0 * L:1 * L]
    Wn_ex = proc_Wn[1 * L:2 * L]
    Wn_agg = proc_Wn[2 * L:3 * L]

    bd = functools.partial(_blockdiag, k=4)
    be4 = lambda b: jnp.tile(b, 4)

    ea4 = edge_attr.reshape(E4, 4 * DE)

    (base1,) = _dense(
        [ea4],
        [([("in", 0, _blockdiag(We_e, 4))], be4(enc_be), False)],
        8000,
    )
    S1, R1 = _dense(
        [x],
        [
            ([("in", 0, We_s)], jnp.zeros((L,), jnp.float32), False),
            ([("in", 0, We_r)], jnp.zeros((L,), jnp.float32), False),
        ],
        2000,
    )

    he3, agg1p = _sc_block_we(s3, r3, base1.reshape(NW, EW, L), S1, R1, zeros)
    he4 = he3.reshape(E4, 128)

    (base2,) = _dense(
        [he4],
        [([("in", 0, bd(W_ce + W_ee))], be4(proc_be), False)],
        8000,
    )
    hx, S2, R2 = _dense(
        [x, agg1p[0, :N], agg1p[1, :N]],
        [
            (
                [("in", 0, enc_Wn[:DF]), ("in", 1, enc_Wn[DF:]),
                 ("in", 2, enc_Wn[DF:])],
                enc_bn, True,
            ),
            ([("out", 0, W_scx + W_sex)], jnp.zeros((L,), jnp.float32), False),
            ([("out", 0, W_rcx + W_rex)], jnp.zeros((L,), jnp.float32), False),
        ],
        2000,
    )

    ce13, agg2p = _sc_block_we(s3, r3, base2.reshape(NW, EW, L), S2, R2, zeros)
    ce14 = ce13.reshape(E4, 128)

    (base3,) = _dense(
        [ce14, he4],
        [([("in", 0, bd(W_ce)), ("in", 1, bd(W_ee))], be4(proc_be), False)],
        8000,
    )
    cx1, S3, R3 = _dense(
        [hx, agg2p[0, :N], agg2p[1, :N]],
        [
            (
                [("in", 0, Wn_cx + Wn_ex), ("in", 1, Wn_agg), ("in", 2, Wn_agg)],
                proc_bn, True,
            ),
            (
                [("out", 0, W_scx), ("in", 0, W_sex)],
                jnp.zeros((L,), jnp.float32), False,
            ),
            (
                [("out", 0, W_rcx), ("in", 0, W_rex)],
                jnp.zeros((L,), jnp.float32), False,
            ),
        ],
        2000,
    )

    ce23, agg3p = _sc_block_we(s3, r3, base3.reshape(NW, EW, L), S3, R3, zeros)
    ce24 = ce23.reshape(E4, 128)

    (base4,) = _dense(
        [ce24],
        [([("in", 0, bd(dec_We[:L]))], be4(dec_be), False)],
        8000,
    )
    cx2, S4, R4 = _dense(
        [cx1, hx, agg3p[0, :N], agg3p[1, :N]],
        [
            (
                [("in", 0, Wn_cx), ("in", 1, Wn_ex), ("in", 2, Wn_agg),
                 ("in", 3, Wn_agg)],
                proc_bn, True,
            ),
            ([("out", 0, dec_We[L:2 * L])], jnp.zeros((L,), jnp.float32), False),
            ([("out", 0, dec_We[2 * L:])], jnp.zeros((L,), jnp.float32), False),
        ],
        2000,
    )

    (agg4p,) = _sc_block_agg(s3, r3, base4.reshape(NW, EW, L), S4, R4, zeros)

    (out_x,) = _dense(
        [cx2, agg4p[0, :N], agg4p[1, :N]],
        [
            (
                [("in", 0, dec_Wn[:L]), ("in", 1, dec_Wn[L:]),
                 ("in", 2, dec_Wn[L:])],
                dec_bn, True,
            ),
        ],
        2000,
    )
    return out_x


# speedup vs baseline: 6.9596x; 1.7232x over previous
import functools

import jax
import jax.numpy as jnp
from jax import lax
from jax.experimental import pallas as pl
from jax.experimental.pallas import tpu as pltpu
from jax.experimental.pallas import tpu_sc as plsc

N = 10000
E = 320000
DF = 128
DE = 16
L = 32

NC = 2
NS = 16
NW = NC * NS
EW = E // NW
CH = 80
NCH = EW // CH
NP = 10240
NROWS = NP // NS

E4 = E // 4


def _make_sc_block(write_e: bool):
    mesh = plsc.VectorSubcoreMesh(
        core_axis_name="c", subcore_axis_name="s", num_cores=NC, num_subcores=NS
    )
    out_type = []
    if write_e:
        out_type.append(jax.ShapeDtypeStruct((NW, EW, L), jnp.float32))
    out_type.append(jax.ShapeDtypeStruct((NC, NP, L), jnp.float32))

    scratch = [
        pltpu.VMEM((NCH, CH), jnp.int32),
        pltpu.VMEM((NCH, CH), jnp.int32),
        pltpu.VMEM((2, CH, L), jnp.float32),
        pltpu.VMEM((2, CH, L), jnp.float32),
        pltpu.VMEM((2, CH, L), jnp.float32),
        pltpu.VMEM((2, CH, L), jnp.float32),
        pltpu.SemaphoreType.DMA,
        pltpu.SemaphoreType.DMA,
        pltpu.SemaphoreType.DMA,
        pltpu.SemaphoreType.DMA,
        pltpu.VMEM_SHARED((NP, L), jnp.float32),
    ]

    def body(s_hbm, r_hbm, base_hbm, S_hbm, R_hbm, z_hbm, *rest):
        if write_e:
            e_out, agg_out = rest[0], rest[1]
            scr = rest[2:]
        else:
            agg_out = rest[0]
            scr = rest[1:]
        s_v, r_v, b_v, sr_v, rr_v, e_v, sem_b, sem_g, sem_sl, sem_si, agg_sh = scr

        c = lax.axis_index("c")
        sid = lax.axis_index("s")
        t = c * NS + sid

        pltpu.sync_copy(
            z_hbm.at[pl.ds(sid * NROWS, NROWS)],
            agg_sh.at[pl.ds(sid * NROWS, NROWS)],
        )
        pltpu.sync_copy(s_hbm.at[t], s_v)
        pltpu.sync_copy(r_hbm.at[t], r_v)
        plsc.subcore_barrier()

        def issue_in(j, b):
            pltpu.async_copy(base_hbm.at[t, pl.ds(j * CH, CH)], b_v.at[b], sem_b)
            pltpu.async_copy(S_hbm.at[s_v.at[j]], sr_v.at[b], sem_g)
            pltpu.async_copy(R_hbm.at[r_v.at[j]], rr_v.at[b], sem_g)

        def wait_in(j, b):
            pltpu.make_async_copy(
                base_hbm.at[t, pl.ds(j * CH, CH)], b_v.at[b], sem_b).wait()
            pltpu.make_async_copy(S_hbm.at[s_v.at[j]], sr_v.at[b], sem_g).wait()
            pltpu.make_async_copy(R_hbm.at[r_v.at[j]], rr_v.at[b], sem_g).wait()

        def drain_store(j, b):
            if write_e:
                pltpu.make_async_copy(
                    e_v.at[b], e_out.at[t, pl.ds(j * CH, CH)], sem_sl).wait()
            pltpu.make_async_copy(
                e_v.at[b], agg_sh.at[r_v.at[j]], sem_si).wait()

        def compute(j, b):
            @pl.loop(0, CH, unroll=8)
            def _row(i):
                for h in range(2):
                    sl = pl.ds(h * 16, 16)
                    e_v[b, i, sl] = jnp.maximum(
                        b_v[b, i, sl] + sr_v[b, i, sl] + rr_v[b, i, sl], 0.0
                    )

        def issue_store(j, b):
            if write_e:
                pltpu.async_copy(
                    e_v.at[b], e_out.at[t, pl.ds(j * CH, CH)], sem_sl)
            pltpu.async_copy(e_v.at[b], agg_sh.at[r_v.at[j]], sem_si, add=True)

        issue_in(0, 0)

        @pl.loop(0, NCH // 2)
        def _pair(jj):
            for par in range(2):
                j = 2 * jj + par
                nb = 1 - par
                issue_in(j + 1, nb)
                wait_in(j, par)

                @pl.when(j >= 2)
                def _():
                    drain_store(j, par)

                compute(j, par)
                issue_store(j, par)

        jl = NCH - 1
        wait_in(jl, 0)
        drain_store(jl - 2, 0)
        compute(jl, 0)
        issue_store(jl, 0)
        drain_store(jl - 1, 1)
        drain_store(jl, 0)

        plsc.subcore_barrier()
        pltpu.sync_copy(
            agg_sh.at[pl.ds(sid * NROWS, NROWS)],
            agg_out.at[c, pl.ds(sid * NROWS, NROWS)],
        )

    return pl.kernel(
        body,
        out_type=tuple(out_type),
        mesh=mesh,
        scratch_types=scratch,
        compiler_params=pltpu.CompilerParams(use_tc_tiling_on_sc=False),
    )


_sc_block_we = _make_sc_block(True)
_sc_block_agg = _make_sc_block(False)


def _dense(ins, outs_spec, block_rows):
    rows = ins[0].shape[0]
    grid = rows // block_rows
    n_in = len(ins)
    w_arrays = []
    w_pos = []
    for terms, _, _ in outs_spec:
        pos = []
        for _, _, W in terms:
            pos.append(len(w_arrays))
            w_arrays.append(W)
        w_pos.append(pos)
    biases = [b.reshape(1, -1) for _, b, _ in outs_spec]
    n_w = len(w_arrays)
    n_out = len(outs_spec)

    def body(*refs):
        in_refs = refs[:n_in]
        w_refs = refs[n_in:n_in + n_w]
        b_refs = refs[n_in + n_w:n_in + n_w + n_out]
        o_refs = refs[n_in + n_w + n_out:]
        outvals = []
        for j, (terms, _, relu) in enumerate(outs_spec):
            acc = b_refs[j][...]
            for (kind, idx, _), wp in zip(terms, w_pos[j]):
                op = in_refs[idx][...] if kind == "in" else outvals[idx]
                acc = acc + jnp.dot(
                    op, w_refs[wp][...],
                    preferred_element_type=jnp.float32,
                    precision=jax.lax.Precision.HIGHEST,
                )
            val = jnp.maximum(acc, 0.0) if relu else acc
            outvals.append(val)
            o_refs[j][...] = val

    in_specs = (
        [pl.BlockSpec((block_rows, a.shape[1]), lambda i: (i, 0)) for a in ins]
        + [pl.BlockSpec(w.shape, lambda i: (0, 0)) for w in w_arrays]
        + [pl.BlockSpec(b.shape, lambda i: (0, 0)) for b in biases]
    )
    out_specs = [
        pl.BlockSpec((block_rows, b.shape[1]), lambda i: (i, 0)) for b in biases
    ]
    out_shape = [
        jax.ShapeDtypeStruct((rows, b.shape[1]), jnp.float32) for b in biases
    ]
    res = pl.pallas_call(
        body,
        grid=(grid,),
        in_specs=in_specs,
        out_specs=out_specs,
        out_shape=out_shape,
    )(*ins, *w_arrays, *biases)
    return res


def _blockdiag(W, k):
    din, dout = W.shape
    out = jnp.zeros((k * din, k * dout), jnp.float32)
    for i in range(k):
        out = out.at[i * din:(i + 1) * din, i * dout:(i + 1) * dout].set(W)
    return out


def kernel(x, edge_attr, edge_index,
           enc_We, enc_be, enc_Wn, enc_bn,
           proc_We, proc_be, proc_Wn, proc_bn,
           dec_We, dec_be, dec_Wn, dec_bn):
    s3 = edge_index[0].reshape(NW, NCH, CH)
    r3 = edge_index[1].reshape(NW, NCH, CH)
    zeros = jnp.zeros((NP, L), jnp.float32)

    We_e = enc_We[:DE]
    We_s = enc_We[DE:DE + DF]
    We_r = enc_We[DE + DF:]
    W_ce = proc_We[0 * L:1 * L]
    W_ee = proc_We[1 * L:2 * L]
    W_scx = proc_We[2 * L:3 * L]
    W_sex = proc_We[3 * L:4 * L]
    W_rcx = proc_We[4 * L:5 * L]
    W_rex = proc_We[5 * L:6 * L]
    Wn_cx = proc_Wn[0 * L:1 * L]
    Wn_ex = proc_Wn[1 * L:2 * L]
    Wn_agg = proc_Wn[2 * L:3 * L]

    bd = functools.partial(_blockdiag, k=4)
    be4 = lambda b: jnp.tile(b, 4)

    ea4 = edge_attr.reshape(E4, 4 * DE)

    (base1,) = _dense(
        [ea4],
        [([("in", 0, _blockdiag(We_e, 4))], be4(enc_be), False)],
        8000,
    )
    S1, R1 = _dense(
        [x],
        [
            ([("in", 0, We_s)], jnp.zeros((L,), jnp.float32), False),
            ([("in", 0, We_r)], jnp.zeros((L,), jnp.float32), False),
        ],
        2000,
    )

    he3, agg1p = _sc_block_we(s3, r3, base1.reshape(NW, EW, L), S1, R1, zeros)
    he4 = he3.reshape(E4, 128)

    (base2,) = _dense(
        [he4],
        [([("in", 0, bd(W_ce + W_ee))], be4(proc_be), False)],
        8000,
    )
    hx, S2, R2 = _dense(
        [x, agg1p[0, :N], agg1p[1, :N]],
        [
            (
                [("in", 0, enc_Wn[:DF]), ("in", 1, enc_Wn[DF:]),
                 ("in", 2, enc_Wn[DF:])],
                enc_bn, True,
            ),
            ([("out", 0, W_scx + W_sex)], jnp.zeros((L,), jnp.float32), False),
            ([("out", 0, W_rcx + W_rex)], jnp.zeros((L,), jnp.float32), False),
        ],
        2000,
    )

    ce13, agg2p = _sc_block_we(s3, r3, base2.reshape(NW, EW, L), S2, R2, zeros)
    ce14 = ce13.reshape(E4, 128)

    (base3,) = _dense(
        [ce14, he4],
        [([("in", 0, bd(W_ce)), ("in", 1, bd(W_ee))], be4(proc_be), False)],
        8000,
    )
    cx1, S3, R3 = _dense(
        [hx, agg2p[0, :N], agg2p[1, :N]],
        [
            (
                [("in", 0, Wn_cx + Wn_ex), ("in", 1, Wn_agg), ("in", 2, Wn_agg)],
                proc_bn, True,
            ),
            (
                [("out", 0, W_scx), ("in", 0, W_sex)],
                jnp.zeros((L,), jnp.float32), False,
            ),
            (
                [("out", 0, W_rcx), ("in", 0, W_rex)],
                jnp.zeros((L,), jnp.float32), False,
            ),
        ],
        2000,
    )

    ce23, agg3p = _sc_block_we(s3, r3, base3.reshape(NW, EW, L), S3, R3, zeros)
    ce24 = ce23.reshape(E4, 128)

    (base4,) = _dense(
        [ce24],
        [([("in", 0, bd(dec_We[:L]))], be4(dec_be), False)],
        8000,
    )
    cx2, S4, R4 = _dense(
        [cx1, hx, agg3p[0, :N], agg3p[1, :N]],
        [
            (
                [("in", 0, Wn_cx), ("in", 1, Wn_ex), ("in", 2, Wn_agg),
                 ("in", 3, Wn_agg)],
                proc_bn, True,
            ),
            ([("out", 0, dec_We[L:2 * L])], jnp.zeros((L,), jnp.float32), False),
            ([("out", 0, dec_We[2 * L:])], jnp.zeros((L,), jnp.float32), False),
        ],
        2000,
    )

    (agg4p,) = _sc_block_agg(s3, r3, base4.reshape(NW, EW, L), S4, R4, zeros)

    (out_x,) = _dense(
        [cx2, agg4p[0, :N], agg4p[1, :N]],
        [
            (
                [("in", 0, dec_Wn[:L]), ("in", 1, dec_Wn[L:]),
                 ("in", 2, dec_Wn[L:])],
                dec_bn, True,
            ),
        ],
        2000,
    )
    return out_x
